# baseline (speedup 1.0000x reference)
import functools

import jax
import jax.numpy as jnp
from jax import lax
from jax.experimental import pallas as pl
from jax.experimental.pallas import tpu as pltpu
from jax.experimental.pallas import tpu_sc as plsc

N = 10000
NP = 10240
CH = 128
HALF = CH // 2
E = 320000
NS = 16
C = 80
NCHUNK = E // NS // C
RPT = NP // NS
NSUB = 5
SUB = RPT // NSUB
HR = NP // 128
HRT = HR // NS
BR = 1000


def _tc_linear(x, w_lin):
    def body(x_ref, w_ref, o0_ref, o1_ref):
        r = lax.dot_general(x_ref[...], w_ref[...], (((1,), (1,)), ((), ())),
                            preferred_element_type=jnp.float32)
        o0_ref[...] = r[:, :HALF]
        o1_ref[...] = r[:, HALF:]

    return pl.pallas_call(
        body,
        grid=(N // BR,),
        in_specs=[pl.BlockSpec((BR, CH), lambda i: (i, 0)),
                  pl.BlockSpec((CH, CH), lambda i: (0, 0))],
        out_specs=[pl.BlockSpec((BR, HALF), lambda i: (i, 0)),
                   pl.BlockSpec((BR, HALF), lambda i: (i, 0))],
        out_shape=[jax.ShapeDtypeStruct((N, HALF), jnp.float32)] * 2,
    )(x, w_lin)


def _tc_mlp(o0, o1, d_col, b_conv, w1, b1, w2, b2):
    def body(o0_ref, o1_ref, d_ref, bc_ref, w1_ref, b1_ref, w2_ref, b2_ref,
             y_ref):
        d = d_ref[...]
        dinv = jnp.where(d > 0.0, 1.0 / d, 0.0)
        t = jnp.concatenate([o0_ref[...], o1_ref[...]], axis=1)
        t = t * dinv + bc_ref[...]
        h = lax.dot_general(t, w1_ref[...], (((1,), (1,)), ((), ())),
                            preferred_element_type=jnp.float32) + b1_ref[...]
        h = jnp.maximum(h, 0.0)
        y = lax.dot_general(h, w2_ref[...], (((1,), (1,)), ((), ())),
                            preferred_element_type=jnp.float32) + b2_ref[...]
        y_ref[...] = jnp.tanh(y)

    return pl.pallas_call(
        body,
        grid=(N // BR,),
        in_specs=[pl.BlockSpec((BR, HALF), lambda i: (i, 0)),
                  pl.BlockSpec((BR, HALF), lambda i: (i, 0)),
                  pl.BlockSpec((BR, 1), lambda i: (i, 0)),
                  pl.BlockSpec((1, CH), lambda i: (0, 0)),
                  pl.BlockSpec((CH, CH), lambda i: (0, 0)),
                  pl.BlockSpec((1, CH), lambda i: (0, 0)),
                  pl.BlockSpec((CH, CH), lambda i: (0, 0)),
                  pl.BlockSpec((1, CH), lambda i: (0, 0))],
        out_specs=pl.BlockSpec((BR, CH), lambda i: (i, 0)),
        out_shape=jax.ShapeDtypeStruct((N, CH), jnp.float32),
    )(o0, o1, d_col, b_conv.reshape(1, CH), w1, b1.reshape(1, CH), w2,
      b2.reshape(1, CH))


def _sc_conv(xl0, xl1, nidx3, eidx3):
    f32 = jnp.float32
    i32 = jnp.int32
    sds = jax.ShapeDtypeStruct
    mesh = plsc.VectorSubcoreMesh(core_axis_name="c", subcore_axis_name="s",
                                  num_cores=2, num_subcores=NS)

    @functools.partial(
        pl.kernel,
        out_type=(sds((NP, HALF), f32), sds((NP, HALF), f32),
                  sds((NP, HALF), f32), sds((NP, HALF), f32),
                  sds((HR, 128), f32)),
        mesh=mesh,
        scratch_types=[
            pltpu.VMEM_SHARED((NP, HALF), f32),
            pltpu.VMEM_SHARED((HR, 128), f32),
            pltpu.VMEM_SHARED((HR, 128), f32),
            pltpu.VMEM((NCHUNK, C), i32),
            pltpu.VMEM((NCHUNK, C), i32),
            pltpu.VMEM((C, HALF), f32),
            pltpu.VMEM((C, HALF), f32),
            pltpu.VMEM((HR, 128), f32),
            pltpu.VMEM((HR, 128), f32),
            pltpu.VMEM((SUB, HALF), f32),
            pltpu.VMEM((1, 128), f32),
            pltpu.VMEM((1, HR), i32),
            pltpu.SemaphoreType.DMA,
            pltpu.SemaphoreType.DMA,
            pltpu.SemaphoreType.DMA,
            pltpu.SemaphoreType.DMA,
        ],
        compiler_params=pltpu.CompilerParams(use_tc_tiling_on_sc=False,
                                             needs_layout_passes=False),
    )
    def k(xl0_hbm, xl1_hbm, nidx_hbm, eidx_hbm,
          ef0_hbm, ef1_hbm, out0_hbm, out1_hbm, dd_hbm,
          acc_sh, hb_sh, hd_sh,
          eidx_v, nidx_v, rows0, rows1, hb_v, hd_v, big_v, hrow_v, iden_v,
          gsem0, gsem1, ssem0, ssem1):
        cid = lax.axis_index("c")
        sid = lax.axis_index("s")
        base = sid * RPT
        zero16 = jnp.zeros((16,), f32)
        ones16 = jnp.ones((16,), f32)
        lane = lax.iota(i32, 16)

        def zrow128(ref):
            def zb(r, _):
                for jj in range(8):
                    ref[r, pl.ds(jj * 16, 16)] = zero16
                return 0
            lax.fori_loop(0, HR, zb, 0)
        zrow128(hb_v)
        zrow128(hd_v)

        def z64(r, _):
            for jj in range(HALF // 16):
                big_v[r, pl.ds(jj * 16, 16)] = zero16
            return 0
        lax.fori_loop(0, SUB, z64, 0)

        for jj in range(HR // 16):
            iden_v[0, pl.ds(jj * 16, 16)] = lane + jj * 16
        for jj in range(8):
            hrow_v[0, pl.ds(jj * 16, 16)] = zero16

        for h in range(NSUB):
            hb = base + h * SUB
            pltpu.sync_copy(big_v, acc_sh.at[pl.ds(hb, SUB)])
        pltpu.sync_copy(hrow_v, hb_sh.at[pl.ds(sid * HRT, 1)])
        pltpu.sync_copy(hrow_v, hd_sh.at[pl.ds(sid * HRT, 1)])
        for r in range(1, HRT):
            pltpu.sync_copy(hrow_v, hb_sh.at[pl.ds(sid * HRT + r, 1)])
            pltpu.sync_copy(hrow_v, hd_sh.at[pl.ds(sid * HRT + r, 1)])
        pltpu.sync_copy(eidx_hbm.at[sid], eidx_v)
        pltpu.sync_copy(nidx_hbm.at[sid], nidx_v)
        plsc.subcore_barrier()

        def run_pass(table_hbm, src_idx, dst_idx, with_counts):
            bufs = ((rows0, gsem0, ssem0), (rows1, gsem1, ssem1))
            for b in range(2):
                rb, gs, _ = bufs[b]
                pltpu.async_copy(table_hbm.at[src_idx.at[b]], rb, gs)

            def body(jj, _):
                for b in range(2):
                    rb, gs, ss = bufs[b]
                    j = jj * 2 + b
                    pltpu.make_async_copy(
                        table_hbm.at[src_idx.at[j]], rb, gs).wait()
                    pltpu.async_copy(rb, acc_sh.at[dst_idx.at[j]], ss,
                                     add=True)
                    if with_counts:
                        def cnt(kk, _):
                            ev = eidx_v[j, pl.ds(kk * 16, 16)]
                            plsc.addupdate_scatter(
                                hb_v, [lax.shift_right_logical(ev, 7),
                                       lax.bitwise_and(ev, 127)], ones16)
                            nv = nidx_v[j, pl.ds(kk * 16, 16)]
                            plsc.addupdate_scatter(
                                hd_v, [lax.shift_right_logical(nv, 7),
                                       lax.bitwise_and(nv, 127)], ones16)
                            return 0
                        lax.fori_loop(0, C // 16, cnt, 0)
                    pltpu.make_async_copy(
                        rb, acc_sh.at[dst_idx.at[j]], ss).wait()

                    @pl.when(j + 2 < NCHUNK)
                    def _():
                        pltpu.async_copy(
                            table_hbm.at[src_idx.at[j + 2]], rb, gs)
                return 0
            lax.fori_loop(0, NCHUNK // 2, body, 0)

        @pl.when(cid == 0)
        def _():
            run_pass(xl0_hbm, nidx_v, eidx_v, True)

        @pl.when(cid == 1)
        def _():
            run_pass(xl1_hbm, nidx_v, eidx_v, True)

        pltpu.sync_copy(hb_v, hb_sh.at[iden_v.at[0]], add=True)
        pltpu.sync_copy(hd_v, hd_sh.at[iden_v.at[0]], add=True)
        plsc.subcore_barrier()

        for h in range(NSUB):
            hb = base + h * SUB
            pltpu.sync_copy(acc_sh.at[pl.ds(hb, SUB)], big_v)
            pltpu.sync_copy(hb_sh.at[pl.ds(sid * HRT + h, 1)], hrow_v)

            def grp(g, _):
                bv = hrow_v[0, pl.ds(g * 16, 16)]
                s = jnp.where(bv > 0.0, 1.0 / bv, 0.0)
                ridx = lane + g * 16

                def col(cc, _):
                    ci = jnp.full((16,), cc, i32)
                    v = plsc.load_gather(big_v, [ridx, ci])
                    plsc.store_scatter(big_v, [ridx, ci], v * s)
                    return 0
                lax.fori_loop(0, HALF, col, 0)
                return 0
            lax.fori_loop(0, SUB // 16, grp, 0)

            @pl.when(cid == 0)
            def _():
                pltpu.sync_copy(big_v, ef0_hbm.at[pl.ds(hb, SUB)])

            @pl.when(cid == 1)
            def _():
                pltpu.sync_copy(big_v, ef1_hbm.at[pl.ds(hb, SUB)])

        def z64b(r, _):
            for jj in range(HALF // 16):
                big_v[r, pl.ds(jj * 16, 16)] = zero16
            return 0
        lax.fori_loop(0, SUB, z64b, 0)
        for h in range(NSUB):
            hb = base + h * SUB
            pltpu.sync_copy(big_v, acc_sh.at[pl.ds(hb, SUB)])
        plsc.subcore_barrier()

        @pl.when(cid == 0)
        def _():
            run_pass(ef0_hbm, eidx_v, nidx_v, False)

        @pl.when(cid == 1)
        def _():
            run_pass(ef1_hbm, eidx_v, nidx_v, False)

        plsc.subcore_barrier()

        @pl.when(cid == 0)
        def _():
            pltpu.sync_copy(acc_sh.at[pl.ds(base, RPT)],
                            out0_hbm.at[pl.ds(base, RPT)])
            pltpu.sync_copy(hd_sh.at[pl.ds(sid * HRT, HRT)],
                            dd_hbm.at[pl.ds(sid * HRT, HRT)])

        @pl.when(cid == 1)
        def _():
            pltpu.sync_copy(acc_sh.at[pl.ds(base, RPT)],
                            out1_hbm.at[pl.ds(base, RPT)])

    return k(xl0, xl1, nidx3, eidx3)


def kernel(x, edge_index, W_lin, b_conv, W1, b1, W2, b2):
    nidx = edge_index[0].astype(jnp.int32).reshape(NS, NCHUNK, C)
    eidx = edge_index[1].astype(jnp.int32).reshape(NS, NCHUNK, C)
    xl0, xl1 = _tc_linear(x, W_lin)
    _, _, o0, o1, dd = _sc_conv(xl0, xl1, nidx, eidx)
    d_col = dd.reshape(NP)[:N].reshape(N, 1)
    return _tc_mlp(o0, o1, d_col, b_conv, W1, b1, W2, b2)


# speedup vs baseline: 1.1281x; 1.1281x over previous
import functools

import jax
import jax.numpy as jnp
from jax import lax
from jax.experimental import pallas as pl
from jax.experimental.pallas import tpu as pltpu
from jax.experimental.pallas import tpu_sc as plsc

N = 10000
NP = 10240
CH = 128
HALF = CH // 2
E = 320000
NS = 16
C = 128
EP = 327680
NCHUNK = EP // NS // C
RPT = NP // NS
NSUB = 5
SUB = RPT // NSUB
HR = NP // 128
HRT = HR // NS
BR = 1024


def _tc_linear(x, w_lin):
    def body(x_ref, w_ref, o0_ref, o1_ref):
        r = lax.dot_general(x_ref[...], w_ref[...], (((1,), (1,)), ((), ())),
                            preferred_element_type=jnp.float32)
        o0_ref[...] = r[:, :HALF]
        o1_ref[...] = r[:, HALF:]

    return pl.pallas_call(
        body,
        grid=(NP // BR,),
        in_specs=[pl.BlockSpec((BR, CH), lambda i: (i, 0)),
                  pl.BlockSpec((CH, CH), lambda i: (0, 0))],
        out_specs=[pl.BlockSpec((BR, HALF), lambda i: (i, 0)),
                   pl.BlockSpec((BR, HALF), lambda i: (i, 0))],
        out_shape=[jax.ShapeDtypeStruct((NP, HALF), jnp.float32)] * 2,
    )(x, w_lin)


def _tc_mlp(o0, o1, d_col, b_conv, w1, b1, w2, b2):
    def body(o0_ref, o1_ref, d_ref, bc_ref, w1_ref, b1_ref, w2_ref, b2_ref,
             y_ref):
        d = d_ref[...]
        dinv = jnp.where(d > 0.0, 1.0 / d, 0.0)
        t = jnp.concatenate([o0_ref[...], o1_ref[...]], axis=1)
        t = t * dinv + bc_ref[...]
        h = lax.dot_general(t, w1_ref[...], (((1,), (1,)), ((), ())),
                            preferred_element_type=jnp.float32) + b1_ref[...]
        h = jnp.maximum(h, 0.0)
        y = lax.dot_general(h, w2_ref[...], (((1,), (1,)), ((), ())),
                            preferred_element_type=jnp.float32) + b2_ref[...]
        y_ref[...] = jnp.tanh(y)

    brm = 1000
    return pl.pallas_call(
        body,
        grid=(N // brm,),
        in_specs=[pl.BlockSpec((brm, HALF), lambda i: (i, 0)),
                  pl.BlockSpec((brm, HALF), lambda i: (i, 0)),
                  pl.BlockSpec((brm, 1), lambda i: (i, 0)),
                  pl.BlockSpec((1, CH), lambda i: (0, 0)),
                  pl.BlockSpec((CH, CH), lambda i: (0, 0)),
                  pl.BlockSpec((1, CH), lambda i: (0, 0)),
                  pl.BlockSpec((CH, CH), lambda i: (0, 0)),
                  pl.BlockSpec((1, CH), lambda i: (0, 0))],
        out_specs=pl.BlockSpec((brm, CH), lambda i: (i, 0)),
        out_shape=jax.ShapeDtypeStruct((N, CH), jnp.float32),
    )(o0, o1, d_col, b_conv.reshape(1, CH), w1, b1.reshape(1, CH), w2,
      b2.reshape(1, CH))


def _sc_conv(xl0, xl1, nidx3, eidx3):
    f32 = jnp.float32
    i32 = jnp.int32
    sds = jax.ShapeDtypeStruct
    mesh = plsc.VectorSubcoreMesh(core_axis_name="c", subcore_axis_name="s",
                                  num_cores=2, num_subcores=NS)

    @functools.partial(
        pl.kernel,
        out_type=(sds((NP, HALF), f32), sds((NP, HALF), f32),
                  sds((NP, HALF), f32), sds((NP, HALF), f32),
                  sds((HR, 128), f32)),
        mesh=mesh,
        scratch_types=[
            pltpu.VMEM_SHARED((NP, HALF), f32),
            pltpu.VMEM_SHARED((HR, 128), f32),
            pltpu.VMEM_SHARED((HR, 128), f32),
            pltpu.VMEM((NCHUNK, C), i32),
            pltpu.VMEM((NCHUNK, C), i32),
            pltpu.VMEM((C, HALF), f32),
            pltpu.VMEM((C, HALF), f32),
            pltpu.VMEM((HR, 128), f32),
            pltpu.VMEM((HR, 128), f32),
            pltpu.VMEM((SUB, HALF), f32),
            pltpu.VMEM((1, 128), f32),
            pltpu.VMEM((1, HR), i32),
            pltpu.SemaphoreType.DMA,
            pltpu.SemaphoreType.DMA,
            pltpu.SemaphoreType.DMA,
            pltpu.SemaphoreType.DMA,
        ],
        compiler_params=pltpu.CompilerParams(use_tc_tiling_on_sc=False,
                                             needs_layout_passes=False),
    )
    def k(xl0_hbm, xl1_hbm, nidx_hbm, eidx_hbm,
          ef0_hbm, ef1_hbm, out0_hbm, out1_hbm, dd_hbm,
          acc_sh, hb_sh, hd_sh,
          eidx_v, nidx_v, rows0, rows1, hb_v, hd_v, big_v, hrow_v, iden_v,
          gsem0, gsem1, ssem0, ssem1):
        cid = lax.axis_index("c")
        sid = lax.axis_index("s")
        base = sid * RPT
        zero16 = jnp.zeros((16,), f32)
        ones16 = jnp.ones((16,), f32)
        lane = lax.iota(i32, 16)

        def zrow128(ref):
            def zb(r, _):
                for jj in range(8):
                    ref[r, pl.ds(jj * 16, 16)] = zero16
                return 0
            lax.fori_loop(0, HR, zb, 0)
        zrow128(hb_v)
        zrow128(hd_v)

        def z64(r, _):
            for jj in range(HALF // 16):
                big_v[r, pl.ds(jj * 16, 16)] = zero16
            return 0
        lax.fori_loop(0, SUB, z64, 0)

        for jj in range(HR // 16):
            iden_v[0, pl.ds(jj * 16, 16)] = lane + jj * 16
        for jj in range(8):
            hrow_v[0, pl.ds(jj * 16, 16)] = zero16

        for h in range(NSUB):
            hb = base + h * SUB
            pltpu.sync_copy(big_v, acc_sh.at[pl.ds(hb, SUB)])
        pltpu.sync_copy(hrow_v, hb_sh.at[pl.ds(sid * HRT, 1)])
        pltpu.sync_copy(hrow_v, hd_sh.at[pl.ds(sid * HRT, 1)])
        for r in range(1, HRT):
            pltpu.sync_copy(hrow_v, hb_sh.at[pl.ds(sid * HRT + r, 1)])
            pltpu.sync_copy(hrow_v, hd_sh.at[pl.ds(sid * HRT + r, 1)])
        pltpu.sync_copy(eidx_hbm.at[sid], eidx_v)
        pltpu.sync_copy(nidx_hbm.at[sid], nidx_v)
        plsc.subcore_barrier()

        def run_pass(table_hbm, src_idx, dst_idx, with_counts):
            bufs = ((rows0, gsem0, ssem0), (rows1, gsem1, ssem1))
            for b in range(2):
                rb, gs, _ = bufs[b]
                pltpu.async_copy(table_hbm.at[src_idx.at[b]], rb, gs)

            def body(jj, _):
                for b in range(2):
                    rb, gs, ss = bufs[b]
                    j = jj * 2 + b
                    pltpu.make_async_copy(
                        table_hbm.at[src_idx.at[j]], rb, gs).wait()
                    pltpu.async_copy(rb, acc_sh.at[dst_idx.at[j]], ss,
                                     add=True)
                    if with_counts:
                        def cnt(kk, _):
                            ev = eidx_v[j, pl.ds(kk * 16, 16)]
                            plsc.addupdate_scatter(
                                hb_v, [lax.shift_right_logical(ev, 7),
                                       lax.bitwise_and(ev, 127)], ones16)
                            nv = nidx_v[j, pl.ds(kk * 16, 16)]
                            plsc.addupdate_scatter(
                                hd_v, [lax.shift_right_logical(nv, 7),
                                       lax.bitwise_and(nv, 127)], ones16)
                            return 0
                        lax.fori_loop(0, C // 16, cnt, 0)
                    pltpu.make_async_copy(
                        rb, acc_sh.at[dst_idx.at[j]], ss).wait()

                    @pl.when(j + 2 < NCHUNK)
                    def _():
                        pltpu.async_copy(
                            table_hbm.at[src_idx.at[j + 2]], rb, gs)
                return 0
            lax.fori_loop(0, NCHUNK // 2, body, 0)

        @pl.when(cid == 0)
        def _():
            run_pass(xl0_hbm, nidx_v, eidx_v, True)

        @pl.when(cid == 1)
        def _():
            run_pass(xl1_hbm, nidx_v, eidx_v, True)

        pltpu.sync_copy(hb_v, hb_sh.at[iden_v.at[0]], add=True)
        pltpu.sync_copy(hd_v, hd_sh.at[iden_v.at[0]], add=True)
        plsc.subcore_barrier()

        for h in range(NSUB):
            hb = base + h * SUB
            pltpu.sync_copy(acc_sh.at[pl.ds(hb, SUB)], big_v)
            pltpu.sync_copy(hb_sh.at[pl.ds(sid * HRT + h, 1)], hrow_v)

            def grp(g, _):
                bv = hrow_v[0, pl.ds(g * 16, 16)]
                s = jnp.where(bv > 0.0, 1.0 / bv, 0.0)
                ridx = lane + g * 16

                def col(cc, _):
                    ci = jnp.full((16,), cc, i32)
                    v = plsc.load_gather(big_v, [ridx, ci])
                    plsc.store_scatter(big_v, [ridx, ci], v * s)
                    return 0
                lax.fori_loop(0, HALF, col, 0)
                return 0
            lax.fori_loop(0, SUB // 16, grp, 0)

            @pl.when(cid == 0)
            def _():
                pltpu.sync_copy(big_v, ef0_hbm.at[pl.ds(hb, SUB)])

            @pl.when(cid == 1)
            def _():
                pltpu.sync_copy(big_v, ef1_hbm.at[pl.ds(hb, SUB)])

        def z64b(r, _):
            for jj in range(HALF // 16):
                big_v[r, pl.ds(jj * 16, 16)] = zero16
            return 0
        lax.fori_loop(0, SUB, z64b, 0)
        for h in range(NSUB):
            hb = base + h * SUB
            pltpu.sync_copy(big_v, acc_sh.at[pl.ds(hb, SUB)])
        plsc.subcore_barrier()

        @pl.when(cid == 0)
        def _():
            run_pass(ef0_hbm, eidx_v, nidx_v, False)

        @pl.when(cid == 1)
        def _():
            run_pass(ef1_hbm, eidx_v, nidx_v, False)

        plsc.subcore_barrier()

        @pl.when(cid == 0)
        def _():
            pltpu.sync_copy(acc_sh.at[pl.ds(base, RPT)],
                            out0_hbm.at[pl.ds(base, RPT)])
            pltpu.sync_copy(hd_sh.at[pl.ds(sid * HRT, HRT)],
                            dd_hbm.at[pl.ds(sid * HRT, HRT)])

        @pl.when(cid == 1)
        def _():
            pltpu.sync_copy(acc_sh.at[pl.ds(base, RPT)],
                            out1_hbm.at[pl.ds(base, RPT)])

    return k(xl0, xl1, nidx3, eidx3)


def kernel(x, edge_index, W_lin, b_conv, W1, b1, W2, b2):
    pad = N + (jnp.arange(EP - E, dtype=jnp.int32) % (NP - N))
    nidx = jnp.concatenate([edge_index[0].astype(jnp.int32), pad])
    eidx = jnp.concatenate([edge_index[1].astype(jnp.int32), pad])
    nidx = nidx.reshape(NS, NCHUNK, C)
    eidx = eidx.reshape(NS, NCHUNK, C)
    x_pad = jnp.pad(x, ((0, NP - N), (0, 0)))
    xl0, xl1 = _tc_linear(x_pad, W_lin)
    _, _, o0, o1, dd = _sc_conv(xl0, xl1, nidx, eidx)
    d_col = dd.reshape(NP)[:N].reshape(N, 1)
    return _tc_mlp(o0, o1, d_col, b_conv, W1, b1, W2, b2)


# speedup vs baseline: 1.2780x; 1.1328x over previous
import functools

import jax
import jax.numpy as jnp
from jax import lax
from jax.experimental import pallas as pl
from jax.experimental.pallas import tpu as pltpu
from jax.experimental.pallas import tpu_sc as plsc

N = 10000
NP = 10240
CH = 128
HALF = CH // 2
E = 320000
NS = 16
C = 128
EP = 327680
NCHUNK = EP // NS // C
RPT = NP // NS
NSUB = 5
SUB = RPT // NSUB
HR = NP // 128
HRT = HR // NS
BR = 1024


def _tc_linear(x, w_lin):
    def body(x_ref, w_ref, o0_ref, o1_ref):
        r = lax.dot_general(x_ref[...], w_ref[...], (((1,), (1,)), ((), ())),
                            preferred_element_type=jnp.float32)
        o0_ref[...] = r[:, :HALF]
        o1_ref[...] = r[:, HALF:]

    return pl.pallas_call(
        body,
        grid=(NP // BR,),
        in_specs=[pl.BlockSpec((BR, CH), lambda i: (i, 0)),
                  pl.BlockSpec((CH, CH), lambda i: (0, 0))],
        out_specs=[pl.BlockSpec((BR, HALF), lambda i: (i, 0)),
                   pl.BlockSpec((BR, HALF), lambda i: (i, 0))],
        out_shape=[jax.ShapeDtypeStruct((NP, HALF), jnp.float32)] * 2,
    )(x, w_lin)


def _tc_mlp(o0, o1, d_col, b_conv, w1, b1, w2, b2):
    def body(o0_ref, o1_ref, d_ref, bc_ref, w1_ref, b1_ref, w2_ref, b2_ref,
             y_ref):
        d = d_ref[...]
        dinv = jnp.where(d > 0.0, 1.0 / d, 0.0)
        t = jnp.concatenate([o0_ref[...], o1_ref[...]], axis=1)
        t = t * dinv + bc_ref[...]
        h = lax.dot_general(t, w1_ref[...], (((1,), (1,)), ((), ())),
                            preferred_element_type=jnp.float32) + b1_ref[...]
        h = jnp.maximum(h, 0.0)
        y = lax.dot_general(h, w2_ref[...], (((1,), (1,)), ((), ())),
                            preferred_element_type=jnp.float32) + b2_ref[...]
        y_ref[...] = jnp.tanh(y)

    brm = 1000
    return pl.pallas_call(
        body,
        grid=(N // brm,),
        in_specs=[pl.BlockSpec((brm, HALF), lambda i: (i, 0)),
                  pl.BlockSpec((brm, HALF), lambda i: (i, 0)),
                  pl.BlockSpec((brm, 1), lambda i: (i, 0)),
                  pl.BlockSpec((1, CH), lambda i: (0, 0)),
                  pl.BlockSpec((CH, CH), lambda i: (0, 0)),
                  pl.BlockSpec((1, CH), lambda i: (0, 0)),
                  pl.BlockSpec((CH, CH), lambda i: (0, 0)),
                  pl.BlockSpec((1, CH), lambda i: (0, 0))],
        out_specs=pl.BlockSpec((brm, CH), lambda i: (i, 0)),
        out_shape=jax.ShapeDtypeStruct((N, CH), jnp.float32),
    )(o0, o1, d_col, b_conv.reshape(1, CH), w1, b1.reshape(1, CH), w2,
      b2.reshape(1, CH))


def _sc_conv(xl0, xl1, nidx3, eidx3):
    f32 = jnp.float32
    i32 = jnp.int32
    sds = jax.ShapeDtypeStruct
    mesh = plsc.VectorSubcoreMesh(core_axis_name="c", subcore_axis_name="s",
                                  num_cores=2, num_subcores=NS)

    @functools.partial(
        pl.kernel,
        out_type=(sds((NP, HALF), f32), sds((NP, HALF), f32),
                  sds((NP, HALF), f32), sds((NP, HALF), f32),
                  sds((HR, 128), f32)),
        mesh=mesh,
        scratch_types=[
            pltpu.VMEM_SHARED((NP, HALF), f32),
            pltpu.VMEM_SHARED((HR, 128), f32),
            pltpu.VMEM_SHARED((HR, 128), f32),
            [pltpu.VMEM((C, HALF), f32) for _ in range(4)],
            [pltpu.VMEM((1, C), i32) for _ in range(8)],
            [pltpu.VMEM((1, C), i32) for _ in range(8)],
            pltpu.VMEM((HR, 128), f32),
            pltpu.VMEM((HR, 128), f32),
            pltpu.VMEM((SUB, HALF), f32),
            pltpu.VMEM((1, 128), f32),
            pltpu.VMEM((1, HR), i32),
            [pltpu.SemaphoreType.DMA for _ in range(4)],
            [pltpu.SemaphoreType.DMA for _ in range(4)],
            [pltpu.SemaphoreType.DMA for _ in range(8)],
        ],
        compiler_params=pltpu.CompilerParams(use_tc_tiling_on_sc=False,
                                             needs_layout_passes=False),
    )
    def k(xl0_hbm, xl1_hbm, nidx_hbm, eidx_hbm,
          ef0_hbm, ef1_hbm, out0_hbm, out1_hbm, dd_hbm,
          acc_sh, hb_sh, hd_sh,
          rows, ei, ni, hb_v, hd_v, big_v, hrow_v, iden_v,
          gsems, ssems, isems):
        cid = lax.axis_index("c")
        sid = lax.axis_index("s")
        base = sid * RPT
        zero16 = jnp.zeros((16,), f32)
        ones16 = jnp.ones((16,), f32)
        lane = lax.iota(i32, 16)

        def zrow128(ref):
            def zb(r, _):
                for jj in range(8):
                    ref[r, pl.ds(jj * 16, 16)] = zero16
                return 0
            lax.fori_loop(0, HR, zb, 0)
        zrow128(hb_v)
        zrow128(hd_v)

        def z64(r, _):
            for jj in range(HALF // 16):
                big_v[r, pl.ds(jj * 16, 16)] = zero16
            return 0
        lax.fori_loop(0, SUB, z64, 0)

        for jj in range(HR // 16):
            iden_v[0, pl.ds(jj * 16, 16)] = lane + jj * 16
        for jj in range(8):
            hrow_v[0, pl.ds(jj * 16, 16)] = zero16

        for h in range(NSUB):
            hb = base + h * SUB
            pltpu.sync_copy(big_v, acc_sh.at[pl.ds(hb, SUB)])
        pltpu.sync_copy(hrow_v, hb_sh.at[pl.ds(sid * HRT, 1)])
        pltpu.sync_copy(hrow_v, hd_sh.at[pl.ds(sid * HRT, 1)])
        for r in range(1, HRT):
            pltpu.sync_copy(hrow_v, hb_sh.at[pl.ds(sid * HRT + r, 1)])
            pltpu.sync_copy(hrow_v, hd_sh.at[pl.ds(sid * HRT + r, 1)])
        plsc.subcore_barrier()

        def run_pass(table_hbm, src_from_n, with_counts):
            row0 = sid * NCHUNK

            def idx_srcdst(q, slot):
                ld = ni[slot] if src_from_n else ei[slot]
                st = ei[slot] if src_from_n else ni[slot]
                return ld, st

            def fire_idx(q, slot):
                pltpu.async_copy(eidx_hbm.at[pl.ds(row0 + q, 1)], ei[slot],
                                 isems[slot])
                pltpu.async_copy(nidx_hbm.at[pl.ds(row0 + q, 1)], ni[slot],
                                 isems[slot])

            def wait_idx(q, slot):
                pltpu.make_async_copy(eidx_hbm.at[pl.ds(row0 + q, 1)],
                                      ei[slot], isems[slot]).wait()
                pltpu.make_async_copy(nidx_hbm.at[pl.ds(row0 + q, 1)],
                                      ni[slot], isems[slot]).wait()

            def fire_gather(q, slot, rslot):
                ld, _ = idx_srcdst(q, slot)
                pltpu.async_copy(table_hbm.at[ld.at[0]], rows[rslot],
                                 gsems[rslot])

            for q in range(6):
                fire_idx(q, q)
            for q in range(2):
                wait_idx(q, q)
                fire_gather(q, q, q)

            def body(jj, _):
                for b in range(8):
                    j = jj * 8 + b
                    rs = b % 4
                    rs2 = (b + 2) % 4
                    is2 = (b + 2) % 8
                    is6 = (b + 6) % 8
                    ld, st = idx_srcdst(j, b)

                    @pl.when(j >= 2)
                    def _():
                        pltpu.make_async_copy(
                            rows[rs2], acc_sh.at[st.at[0]],
                            ssems[rs2]).wait()

                    @pl.when(j + 6 < NCHUNK)
                    def _():
                        fire_idx(j + 6, is6)

                    @pl.when(j + 2 < NCHUNK)
                    def _():
                        wait_idx(j + 2, is2)
                        fire_gather(j + 2, is2, rs2)

                    pltpu.make_async_copy(table_hbm.at[ld.at[0]], rows[rs],
                                          gsems[rs]).wait()
                    pltpu.async_copy(rows[rs], acc_sh.at[st.at[0]],
                                     ssems[rs], add=True)
                    if with_counts:
                        def cnt(kk, _):
                            ev = ei[b][0, pl.ds(kk * 16, 16)]
                            plsc.addupdate_scatter(
                                hb_v, [lax.shift_right_logical(ev, 7),
                                       lax.bitwise_and(ev, 127)], ones16)
                            nv = ni[b][0, pl.ds(kk * 16, 16)]
                            plsc.addupdate_scatter(
                                hd_v, [lax.shift_right_logical(nv, 7),
                                       lax.bitwise_and(nv, 127)], ones16)
                            return 0
                        lax.fori_loop(0, C // 16, cnt, 0)
                return 0
            lax.fori_loop(0, NCHUNK // 8, body, 0)

            for q in (NCHUNK - 2, NCHUNK - 1):
                slot = q % 8
                rs = q % 4
                _, st = idx_srcdst(q, slot)
                pltpu.make_async_copy(rows[rs], acc_sh.at[st.at[0]],
                                     ssems[rs]).wait()

        @pl.when(cid == 0)
        def _():
            run_pass(xl0_hbm, True, True)

        @pl.when(cid == 1)
        def _():
            run_pass(xl1_hbm, True, True)

        pltpu.sync_copy(hb_v, hb_sh.at[iden_v.at[0]], add=True)
        pltpu.sync_copy(hd_v, hd_sh.at[iden_v.at[0]], add=True)
        plsc.subcore_barrier()

        for h in range(NSUB):
            hb = base + h * SUB
            pltpu.sync_copy(acc_sh.at[pl.ds(hb, SUB)], big_v)
            pltpu.sync_copy(hb_sh.at[pl.ds(sid * HRT + h, 1)], hrow_v)

            def grp(g, _):
                bv = hrow_v[0, pl.ds(g * 16, 16)]
                s = jnp.where(bv > 0.0, 1.0 / bv, 0.0)
                ridx = lane + g * 16

                def col(cc, _):
                    ci = jnp.full((16,), cc, i32)
                    v = plsc.load_gather(big_v, [ridx, ci])
                    plsc.store_scatter(big_v, [ridx, ci], v * s)
                    return 0
                lax.fori_loop(0, HALF, col, 0)
                return 0
            lax.fori_loop(0, SUB // 16, grp, 0)

            @pl.when(cid == 0)
            def _():
                pltpu.sync_copy(big_v, ef0_hbm.at[pl.ds(hb, SUB)])

            @pl.when(cid == 1)
            def _():
                pltpu.sync_copy(big_v, ef1_hbm.at[pl.ds(hb, SUB)])

        def z64b(r, _):
            for jj in range(HALF // 16):
                big_v[r, pl.ds(jj * 16, 16)] = zero16
            return 0
        lax.fori_loop(0, SUB, z64b, 0)
        for h in range(NSUB):
            hb = base + h * SUB
            pltpu.sync_copy(big_v, acc_sh.at[pl.ds(hb, SUB)])
        plsc.subcore_barrier()

        @pl.when(cid == 0)
        def _():
            run_pass(ef0_hbm, False, False)

        @pl.when(cid == 1)
        def _():
            run_pass(ef1_hbm, False, False)

        plsc.subcore_barrier()

        @pl.when(cid == 0)
        def _():
            pltpu.sync_copy(acc_sh.at[pl.ds(base, RPT)],
                            out0_hbm.at[pl.ds(base, RPT)])
            pltpu.sync_copy(hd_sh.at[pl.ds(sid * HRT, HRT)],
                            dd_hbm.at[pl.ds(sid * HRT, HRT)])

        @pl.when(cid == 1)
        def _():
            pltpu.sync_copy(acc_sh.at[pl.ds(base, RPT)],
                            out1_hbm.at[pl.ds(base, RPT)])

    return k(xl0, xl1, nidx3, eidx3)


def kernel(x, edge_index, W_lin, b_conv, W1, b1, W2, b2):
    pad = N + (jnp.arange(EP - E, dtype=jnp.int32) % (NP - N))
    nidx = jnp.concatenate([edge_index[0].astype(jnp.int32), pad])
    eidx = jnp.concatenate([edge_index[1].astype(jnp.int32), pad])
    nidx = nidx.reshape(NS * NCHUNK, C)
    eidx = eidx.reshape(NS * NCHUNK, C)
    x_pad = jnp.pad(x, ((0, NP - N), (0, 0)))
    xl0, xl1 = _tc_linear(x_pad, W_lin)
    _, _, o0, o1, dd = _sc_conv(xl0, xl1, nidx, eidx)
    d_col = dd.reshape(NP)[:N].reshape(N, 1)
    return _tc_mlp(o0, o1, d_col, b_conv, W1, b1, W2, b2)


# speedup vs baseline: 1.3368x; 1.0460x over previous
import functools

import jax
import jax.numpy as jnp
from jax import lax
from jax.experimental import pallas as pl
from jax.experimental.pallas import tpu as pltpu
from jax.experimental.pallas import tpu_sc as plsc

N = 10000
NP = 10240
CH = 128
HALF = CH // 2
E = 320000
NS = 16
C = 128
EP = 327680
NCHUNK = EP // NS // C
RPT = NP // NS
NSUB = 5
SUB = RPT // NSUB
HR = NP // 128
HRT = HR // NS
BR = 1024


def _tc_mlp(o0, o1, d_col, w_lin, b_conv, w1, b1, w2, b2):
    def body(o0_ref, o1_ref, d_ref, wl_ref, bc_ref, w1_ref, b1_ref, w2_ref,
             b2_ref, y_ref):
        d = d_ref[...]
        dinv = jnp.where(d > 0.0, 1.0 / d, 0.0)
        agg = jnp.concatenate([o0_ref[...], o1_ref[...]], axis=1) * dinv
        t = lax.dot_general(agg, wl_ref[...], (((1,), (1,)), ((), ())),
                            preferred_element_type=jnp.float32) + bc_ref[...]
        h = lax.dot_general(t, w1_ref[...], (((1,), (1,)), ((), ())),
                            preferred_element_type=jnp.float32) + b1_ref[...]
        h = jnp.maximum(h, 0.0)
        y = lax.dot_general(h, w2_ref[...], (((1,), (1,)), ((), ())),
                            preferred_element_type=jnp.float32) + b2_ref[...]
        y_ref[...] = jnp.tanh(y)

    brm = 1000
    return pl.pallas_call(
        body,
        grid=(N // brm,),
        in_specs=[pl.BlockSpec((brm, HALF), lambda i: (i, 0)),
                  pl.BlockSpec((brm, HALF), lambda i: (i, 0)),
                  pl.BlockSpec((brm, 1), lambda i: (i, 0)),
                  pl.BlockSpec((CH, CH), lambda i: (0, 0)),
                  pl.BlockSpec((1, CH), lambda i: (0, 0)),
                  pl.BlockSpec((CH, CH), lambda i: (0, 0)),
                  pl.BlockSpec((1, CH), lambda i: (0, 0)),
                  pl.BlockSpec((CH, CH), lambda i: (0, 0)),
                  pl.BlockSpec((1, CH), lambda i: (0, 0))],
        out_specs=pl.BlockSpec((brm, CH), lambda i: (i, 0)),
        out_shape=jax.ShapeDtypeStruct((N, CH), jnp.float32),
    )(o0, o1, d_col, w_lin, b_conv.reshape(1, CH), w1, b1.reshape(1, CH), w2,
      b2.reshape(1, CH))


def _sc_conv(xl0, xl1, idx2):
    f32 = jnp.float32
    i32 = jnp.int32
    sds = jax.ShapeDtypeStruct
    mesh = plsc.VectorSubcoreMesh(core_axis_name="c", subcore_axis_name="s",
                                  num_cores=2, num_subcores=NS)

    @functools.partial(
        pl.kernel,
        out_type=(sds((NP, HALF), f32), sds((NP, HALF), f32),
                  sds((NP, HALF), f32), sds((NP, HALF), f32),
                  sds((HR, 128), f32)),
        mesh=mesh,
        scratch_types=[
            pltpu.VMEM_SHARED((NP, HALF), f32),
            pltpu.VMEM_SHARED((HR, 128), f32),
            pltpu.VMEM_SHARED((HR, 128), f32),
            [pltpu.VMEM((C, HALF), f32) for _ in range(4)],
            [pltpu.VMEM((2, C), i32) for _ in range(8)],
            pltpu.VMEM((HR, 128), f32),
            pltpu.VMEM((HR, 128), f32),
            pltpu.VMEM((SUB, HALF), f32),
            pltpu.VMEM((SUB, HALF), f32),
            pltpu.VMEM((1, 128), f32),
            pltpu.VMEM((1, HR), i32),
            [pltpu.SemaphoreType.DMA for _ in range(4)],
            [pltpu.SemaphoreType.DMA for _ in range(4)],
            [pltpu.SemaphoreType.DMA for _ in range(8)],
            pltpu.SemaphoreType.DMA,
        ],
        compiler_params=pltpu.CompilerParams(use_tc_tiling_on_sc=False,
                                             needs_layout_passes=False),
    )
    def k(xl0_hbm, xl1_hbm, idx_hbm,
          ef0_hbm, ef1_hbm, out0_hbm, out1_hbm, dd_hbm,
          acc_sh, hb_sh, hd_sh,
          rows, ib, hb_v, hd_v, big_v, zb_v, hrow_v, iden_v,
          gsems, ssems, isems, zsem):
        cid = lax.axis_index("c")
        sid = lax.axis_index("s")
        base = sid * RPT
        zero16 = jnp.zeros((16,), f32)
        ones16 = jnp.ones((16,), f32)
        lane = lax.iota(i32, 16)

        def zrow128(ref):
            def zb(r, _):
                for jj in range(8):
                    ref[r, pl.ds(jj * 16, 16)] = zero16
                return 0
            lax.fori_loop(0, HR, zb, 0)
        zrow128(hb_v)
        zrow128(hd_v)

        def z64(r, _):
            for jj in range(HALF // 16):
                zb_v[r, pl.ds(jj * 16, 16)] = zero16
            return 0
        lax.fori_loop(0, SUB, z64, 0)

        for jj in range(HR // 16):
            iden_v[0, pl.ds(jj * 16, 16)] = lane + jj * 16
        for jj in range(8):
            hrow_v[0, pl.ds(jj * 16, 16)] = zero16

        for h in range(NSUB):
            hb = base + h * SUB
            pltpu.async_copy(zb_v, acc_sh.at[pl.ds(hb, SUB)], zsem)
        for r in range(HRT):
            pltpu.async_copy(hrow_v, hb_sh.at[pl.ds(sid * HRT + r, 1)], zsem)
            pltpu.async_copy(hrow_v, hd_sh.at[pl.ds(sid * HRT + r, 1)], zsem)
        for h in range(NSUB):
            hb = base + h * SUB
            pltpu.make_async_copy(zb_v, acc_sh.at[pl.ds(hb, SUB)],
                                  zsem).wait()
        for r in range(HRT):
            pltpu.make_async_copy(hrow_v, hb_sh.at[pl.ds(sid * HRT + r, 1)],
                                  zsem).wait()
            pltpu.make_async_copy(hrow_v, hd_sh.at[pl.ds(sid * HRT + r, 1)],
                                  zsem).wait()
        plsc.subcore_barrier()

        def run_pass(table_hbm, src_from_n, with_counts):
            row0 = sid * NCHUNK
            ld_row = 0 if src_from_n else 1
            st_row = 1 - ld_row

            def fire_idx(q, slot):
                pltpu.async_copy(idx_hbm.at[row0 + q], ib[slot], isems[slot])

            def wait_idx(q, slot):
                pltpu.make_async_copy(idx_hbm.at[row0 + q], ib[slot],
                                      isems[slot]).wait()

            def fire_gather(q, slot, rslot):
                pltpu.async_copy(table_hbm.at[ib[slot].at[ld_row]],
                                 rows[rslot], gsems[rslot])

            for q in range(6):
                fire_idx(q, q)
            for q in range(2):
                wait_idx(q, q)
                fire_gather(q, q, q)

            def body(jj, _):
                for b in range(8):
                    j = jj * 8 + b
                    rs = b % 4
                    rs2 = (b + 2) % 4
                    is2 = (b + 2) % 8
                    is6 = (b + 6) % 8
                    st = ib[b].at[st_row]

                    @pl.when(j >= 2)
                    def _():
                        pltpu.make_async_copy(
                            rows[rs2], acc_sh.at[st], ssems[rs2]).wait()

                    @pl.when(j + 6 < NCHUNK)
                    def _():
                        fire_idx(j + 6, is6)

                    @pl.when(j + 2 < NCHUNK)
                    def _():
                        wait_idx(j + 2, is2)
                        fire_gather(j + 2, is2, rs2)

                    pltpu.make_async_copy(table_hbm.at[ib[b].at[ld_row]],
                                          rows[rs], gsems[rs]).wait()
                    pltpu.async_copy(rows[rs], acc_sh.at[st], ssems[rs],
                                     add=True)
                    if with_counts:
                        def cnt(kk, _):
                            ev = ib[b][1, pl.ds(kk * 16, 16)]
                            plsc.addupdate_scatter(
                                hb_v, [lax.shift_right_logical(ev, 7),
                                       lax.bitwise_and(ev, 127)], ones16)
                            nv = ib[b][0, pl.ds(kk * 16, 16)]
                            plsc.addupdate_scatter(
                                hd_v, [lax.shift_right_logical(nv, 7),
                                       lax.bitwise_and(nv, 127)], ones16)
                            return 0
                        lax.fori_loop(0, C // 16, cnt, 0)
                return 0
            lax.fori_loop(0, NCHUNK // 8, body, 0)

            for q in (NCHUNK - 2, NCHUNK - 1):
                rs = q % 4
                pltpu.make_async_copy(rows[rs],
                                      acc_sh.at[ib[q % 8].at[st_row]],
                                      ssems[rs]).wait()

        @pl.when(cid == 0)
        def _():
            run_pass(xl0_hbm, True, True)

        @pl.when(cid == 1)
        def _():
            run_pass(xl1_hbm, True, True)

        pltpu.sync_copy(hb_v, hb_sh.at[iden_v.at[0]], add=True)
        pltpu.sync_copy(hd_v, hd_sh.at[iden_v.at[0]], add=True)
        plsc.subcore_barrier()

        for h in range(NSUB):
            hb = base + h * SUB
            pltpu.sync_copy(acc_sh.at[pl.ds(hb, SUB)], big_v)
            pltpu.sync_copy(hb_sh.at[pl.ds(sid * HRT + h, 1)], hrow_v)
            pltpu.async_copy(zb_v, acc_sh.at[pl.ds(hb, SUB)], zsem)

            def grp(g, _):
                bv = hrow_v[0, pl.ds(g * 16, 16)]
                s = jnp.where(bv > 0.0, 1.0 / bv, 0.0)
                ridx = lane + g * 16

                def col(cc, _):
                    ci = jnp.full((16,), cc, i32)
                    v = plsc.load_gather(big_v, [ridx, ci])
                    plsc.store_scatter(big_v, [ridx, ci], v * s)
                    return 0
                lax.fori_loop(0, HALF, col, 0)
                return 0
            lax.fori_loop(0, SUB // 16, grp, 0)

            @pl.when(cid == 0)
            def _():
                pltpu.sync_copy(big_v, ef0_hbm.at[pl.ds(hb, SUB)])

            @pl.when(cid == 1)
            def _():
                pltpu.sync_copy(big_v, ef1_hbm.at[pl.ds(hb, SUB)])

        for h in range(NSUB):
            hb = base + h * SUB
            pltpu.make_async_copy(zb_v, acc_sh.at[pl.ds(hb, SUB)],
                                  zsem).wait()
        plsc.subcore_barrier()

        @pl.when(cid == 0)
        def _():
            run_pass(ef0_hbm, False, False)

        @pl.when(cid == 1)
        def _():
            run_pass(ef1_hbm, False, False)

        plsc.subcore_barrier()

        @pl.when(cid == 0)
        def _():
            pltpu.sync_copy(acc_sh.at[pl.ds(base, RPT)],
                            out0_hbm.at[pl.ds(base, RPT)])
            pltpu.sync_copy(hd_sh.at[pl.ds(sid * HRT, HRT)],
                            dd_hbm.at[pl.ds(sid * HRT, HRT)])

        @pl.when(cid == 1)
        def _():
            pltpu.sync_copy(acc_sh.at[pl.ds(base, RPT)],
                            out1_hbm.at[pl.ds(base, RPT)])

    return k(xl0, xl1, idx2)


def kernel(x, edge_index, W_lin, b_conv, W1, b1, W2, b2):
    pad = N + (jnp.arange(EP - E, dtype=jnp.int32) % (NP - N))
    nidx = jnp.concatenate([edge_index[0].astype(jnp.int32), pad])
    eidx = jnp.concatenate([edge_index[1].astype(jnp.int32), pad])
    idx2 = jnp.stack([nidx.reshape(NS * NCHUNK, C),
                      eidx.reshape(NS * NCHUNK, C)], axis=1)
    x0 = jnp.pad(x[:, :HALF], ((0, NP - N), (0, 0)))
    x1 = jnp.pad(x[:, HALF:], ((0, NP - N), (0, 0)))
    _, _, o0, o1, dd = _sc_conv(x0, x1, idx2)
    d_col = dd.reshape(NP)[:N].reshape(N, 1)
    return _tc_mlp(o0, o1, d_col, W_lin, b_conv, W1, b1, W2, b2)
